# Initial kernel scaffold; baseline (speedup 1.0000x reference)
import jax
import jax.numpy as jnp
from jax.experimental import pallas as pl


def kernel(x1, adj1, x2, adj2, clm, W11, b11, W12, b12, W21, b21, W22, b22):
    raise NotImplementedError("write your pallas kernel here")


# speedup vs baseline: 1.7754x; 1.7754x over previous
import jax
import jax.numpy as jnp
from jax.experimental import pallas as pl

N = 4096
NFEAT = 256
NHID = 128
TAU = 0.5
BLK = 512


def _sup_body(x1_ref, w1_ref, x2_ref, w2_ref, o1_ref, o2_ref):
    o1_ref[...] = jnp.dot(
        x1_ref[...].astype(jnp.bfloat16), w1_ref[...].astype(jnp.bfloat16),
        preferred_element_type=jnp.float32).astype(jnp.bfloat16)
    o2_ref[...] = jnp.dot(
        x2_ref[...].astype(jnp.bfloat16), w2_ref[...].astype(jnp.bfloat16),
        preferred_element_type=jnp.float32).astype(jnp.bfloat16)


def _mid_body(adj_ref, sup_ref, w2_ref, b1_ref, o_ref):
    h = jnp.dot(adj_ref[...].astype(jnp.bfloat16), sup_ref[...],
                preferred_element_type=jnp.float32)
    h = jnp.maximum(h + b1_ref[...], 0.0).astype(jnp.bfloat16)
    o_ref[...] = jnp.dot(h, w2_ref[...],
                         preferred_element_type=jnp.float32).astype(jnp.bfloat16)


def _out_body(adj_ref, s2_ref, b2_ref, z_ref, zn_ref):
    z = jnp.dot(adj_ref[...].astype(jnp.bfloat16), s2_ref[...],
                preferred_element_type=jnp.float32) + b2_ref[...]
    z_ref[...] = z
    nrm = jnp.sqrt(jnp.sum(z * z, axis=1, keepdims=True))
    zn_ref[...] = (z / nrm).astype(jnp.bfloat16)


def _loss_body(z1n_ref, z2n_ref, clm_ref, acc_ref):
    cos = jax.lax.dot_general(
        z1n_ref[...], z2n_ref[...],
        dimension_numbers=(((1,), (1,)), ((), ())),
        preferred_element_type=jnp.float32)
    sim = jnp.exp(cos * (1.0 / TAU))
    s = jnp.sum(sim, axis=1, keepdims=True)
    w = jnp.sum(sim * clm_ref[...], axis=1, keepdims=True)
    part = jnp.sum(jnp.log(s + 1e-8) - jnp.log(w))

    @pl.when(pl.program_id(0) == 0)
    def _():
        acc_ref[...] = jnp.zeros_like(acc_ref)

    acc_ref[...] = acc_ref[...] + part


def _supports(x1, W11, x2, W21):
    return pl.pallas_call(
        _sup_body,
        out_shape=(
            jax.ShapeDtypeStruct((N, NFEAT), jnp.bfloat16),
            jax.ShapeDtypeStruct((N, NFEAT), jnp.bfloat16),
        ),
    )(x1, W11, x2, W21)


def _mid(adj, sup, W2, b1):
    nblk = N // BLK
    return pl.pallas_call(
        _mid_body,
        grid=(nblk,),
        in_specs=[
            pl.BlockSpec((BLK, N), lambda i: (i, 0)),
            pl.BlockSpec((N, NFEAT), lambda i: (0, 0)),
            pl.BlockSpec((NFEAT, NHID), lambda i: (0, 0)),
            pl.BlockSpec((1, NFEAT), lambda i: (0, 0)),
        ],
        out_specs=pl.BlockSpec((BLK, NHID), lambda i: (i, 0)),
        out_shape=jax.ShapeDtypeStruct((N, NHID), jnp.bfloat16),
    )(adj, sup, W2, b1)


def _outz(adj, s2, b2):
    nblk = N // BLK
    return pl.pallas_call(
        _out_body,
        grid=(nblk,),
        in_specs=[
            pl.BlockSpec((BLK, N), lambda i: (i, 0)),
            pl.BlockSpec((N, NHID), lambda i: (0, 0)),
            pl.BlockSpec((1, NHID), lambda i: (0, 0)),
        ],
        out_specs=(
            pl.BlockSpec((BLK, NHID), lambda i: (i, 0)),
            pl.BlockSpec((BLK, NHID), lambda i: (i, 0)),
        ),
        out_shape=(
            jax.ShapeDtypeStruct((N, NHID), jnp.float32),
            jax.ShapeDtypeStruct((N, NHID), jnp.bfloat16),
        ),
    )(adj, s2, b2)


def _loss(zn1, zn2, clm):
    nblk = N // BLK
    return pl.pallas_call(
        _loss_body,
        grid=(nblk,),
        in_specs=[
            pl.BlockSpec((BLK, NHID), lambda i: (i, 0)),
            pl.BlockSpec((N, NHID), lambda i: (0, 0)),
            pl.BlockSpec((BLK, N), lambda i: (i, 0)),
        ],
        out_specs=pl.BlockSpec((1, 1), lambda i: (0, 0)),
        out_shape=jax.ShapeDtypeStruct((1, 1), jnp.float32),
    )(zn1, zn2, clm)


def kernel(x1, adj1, x2, adj2, clm, W11, b11, W12, b12, W21, b21, W22, b22):
    sup1, sup2 = _supports(x1, W11, x2, W21)
    s21 = _mid(adj1, sup1, W12.astype(jnp.bfloat16), b11.reshape(1, -1))
    s22 = _mid(adj2, sup2, W22.astype(jnp.bfloat16), b21.reshape(1, -1))
    z1, zn1 = _outz(adj1, s21, b12.reshape(1, -1))
    z2, zn2 = _outz(adj2, s22, b22.reshape(1, -1))
    acc = _loss(zn1, zn2, clm)
    cl_loss = (acc[0, 0] / N).astype(jnp.float32).reshape(())
    return (z1, z2, cl_loss)
